# baseline (speedup 1.0000x reference)
import functools

import jax
import jax.numpy as jnp
from jax import lax
from jax.experimental import pallas as pl
from jax.experimental.pallas import tpu as pltpu
from jax.experimental.pallas import tpu_sc as plsc


def _make_gather(V, D, B):
    info = plsc.get_sparse_core_info()
    NC, NS = info.num_cores, info.num_subcores
    NW = NC * NS
    assert B % NW == 0
    b_per_w = B // NW
    C = 800
    assert b_per_w % C == 0
    n_chunks = b_per_w // C

    mesh = plsc.VectorSubcoreMesh(core_axis_name="c", subcore_axis_name="s")

    @functools.partial(
        pl.kernel,
        out_type=jax.ShapeDtypeStruct((B, D), jnp.float32),
        mesh=mesh,
        scratch_types=[
            pltpu.VMEM((C,), jnp.int32),
            pltpu.VMEM((C, D), jnp.float32),
            pltpu.SemaphoreType.DMA,
        ],
        compiler_params=pltpu.CompilerParams(use_tc_tiling_on_sc=False),
    )
    def gather_kernel(idx_hbm, table_hbm, out_hbm, idx_v, rows_v, sem):
        wid = lax.axis_index("s") * NC + lax.axis_index("c")
        base = wid * b_per_w

        def step(g, carry):
            off = base + g * C
            pltpu.sync_copy(idx_hbm.at[pl.ds(off, C)], idx_v)
            pltpu.async_copy(table_hbm.at[idx_v], rows_v, sem).wait()
            pltpu.sync_copy(rows_v, out_hbm.at[pl.ds(off, C)])
            return carry

        lax.fori_loop(0, n_chunks, step, 0)

    return gather_kernel


def kernel(inputs, embed_weight):
    batch, n_tokens = inputs.shape
    V, D = embed_weight.shape
    flat_idx = inputs.reshape(-1).astype(jnp.int32)
    B = flat_idx.shape[0]
    out = _make_gather(V, D, B)(flat_idx, embed_weight)
    return out.reshape(batch, n_tokens, D)


# speedup vs baseline: 1.0124x; 1.0124x over previous
import functools

import jax
import jax.numpy as jnp
from jax import lax
from jax.experimental import pallas as pl
from jax.experimental.pallas import tpu as pltpu
from jax.experimental.pallas import tpu_sc as plsc


def _make_gather(V, D, B):
    info = plsc.get_sparse_core_info()
    NC, NS = info.num_cores, info.num_subcores
    NW = NC * NS
    assert B % NW == 0
    b_per_w = B // NW
    C = 800
    assert b_per_w % (2 * C) == 0
    n_chunks = b_per_w // C
    n_pairs = n_chunks // 2
    assert b_per_w + 2 * C * D <= 131000

    mesh = plsc.VectorSubcoreMesh(core_axis_name="c", subcore_axis_name="s")

    @functools.partial(
        pl.kernel,
        out_type=jax.ShapeDtypeStruct((B, D), jnp.float32),
        mesh=mesh,
        scratch_types=[
            pltpu.VMEM((b_per_w,), jnp.int32),
            pltpu.VMEM((C, D), jnp.float32),
            pltpu.VMEM((C, D), jnp.float32),
            pltpu.SemaphoreType.DMA,
            pltpu.SemaphoreType.DMA,
            pltpu.SemaphoreType.DMA,
            pltpu.SemaphoreType.DMA,
        ],
        compiler_params=pltpu.CompilerParams(use_tc_tiling_on_sc=False),
    )
    def gather_kernel(idx_hbm, table_hbm, out_hbm, idx_v, rows0, rows1,
                      sg0, sg1, so0, so1):
        wid = lax.axis_index("s") * NC + lax.axis_index("c")
        base = wid * b_per_w
        pltpu.sync_copy(idx_hbm.at[pl.ds(base, b_per_w)], idx_v)

        def idx_slice(g):
            return idx_v.at[pl.ds(g * C, C)]

        def out_slice(g):
            return out_hbm.at[pl.ds(base + g * C, C)]

        pltpu.async_copy(table_hbm.at[idx_slice(0)], rows0, sg0)
        pltpu.async_copy(table_hbm.at[idx_slice(1)], rows1, sg1)

        def pair(i, carry):
            g = 2 * i
            pltpu.make_async_copy(table_hbm.at[idx_slice(g)], rows0, sg0).wait()
            pltpu.async_copy(rows0, out_slice(g), so0)
            pltpu.make_async_copy(table_hbm.at[idx_slice(g + 1)], rows1, sg1).wait()
            pltpu.async_copy(rows1, out_slice(g + 1), so1)

            @pl.when(i + 1 < n_pairs)
            def _refill():
                pltpu.make_async_copy(rows0, out_slice(g), so0).wait()
                pltpu.async_copy(table_hbm.at[idx_slice(g + 2)], rows0, sg0)
                pltpu.make_async_copy(rows1, out_slice(g + 1), so1).wait()
                pltpu.async_copy(table_hbm.at[idx_slice(g + 3)], rows1, sg1)

            return carry

        lax.fori_loop(0, n_pairs, pair, 0)
        pltpu.make_async_copy(rows0, out_slice(n_chunks - 2), so0).wait()
        pltpu.make_async_copy(rows1, out_slice(n_chunks - 1), so1).wait()

    return gather_kernel


def kernel(inputs, embed_weight):
    batch, n_tokens = inputs.shape
    V, D = embed_weight.shape
    flat_idx = inputs.reshape(-1).astype(jnp.int32)
    B = flat_idx.shape[0]
    out = _make_gather(V, D, B)(flat_idx, embed_weight)
    return out.reshape(batch, n_tokens, D)
